# baseline (speedup 1.0000x reference)
import functools

import jax
import jax.numpy as jnp
from jax import lax
from jax.experimental import pallas as pl
from jax.experimental.pallas import tpu as pltpu
from jax.experimental.pallas import tpu_sc as plsc

D = 128
NW = 32
CH = 128
TILE_E = 2048
TILE_N = 2000


def _ln_relu(h, g, b):
    m = jnp.mean(h, axis=-1, keepdims=True)
    v = jnp.mean(jnp.square(h - m), axis=-1, keepdims=True)
    return jnp.maximum((h - m) * lax.rsqrt(v + 1e-5) * g + b, 0.0)


def _dot(a, w):
    return jnp.dot(a, w, preferred_element_type=jnp.float32)


def _mlp3_body(x_ref, w1, b1, g1, be1, w2, b2, g2, be2, w3, b3, o_ref):
    h = _dot(x_ref[...], w1[...]) + b1[...]
    h = _ln_relu(h, g1[...], be1[...])
    h = _dot(h, w2[...]) + b2[...]
    h = _ln_relu(h, g2[...], be2[...])
    o_ref[...] = _dot(h, w3[...]) + b3[...]


def _edge_body(ar_ref, bc_ref, e_ref, w1c, b1, g1, be1,
               w2, b2, g2, be2, w3, b3, o_ref):
    e = e_ref[...]
    h = ar_ref[...] + bc_ref[...] + _dot(e, w1c[...]) + b1[...]
    h = _ln_relu(h, g1[...], be1[...])
    h = _dot(h, w2[...]) + b2[...]
    h = _ln_relu(h, g2[...], be2[...])
    o_ref[...] = e + _dot(h, w3[...]) + b3[...]


def _node_body(x_ref, m_ref, w1x, w1m, b1, g1, be1,
               w2, b2, g2, be2, w3, b3, w1a_n, w1b_n,
               o_ref, a_ref, b_ref):
    x = x_ref[...]
    msg = m_ref[0] + m_ref[1]
    h = _dot(x, w1x[...]) + _dot(msg, w1m[...]) + b1[...]
    h = _ln_relu(h, g1[...], be1[...])
    h = _dot(h, w2[...]) + b2[...]
    h = _ln_relu(h, g2[...], be2[...])
    xn = x + _dot(h, w3[...]) + b3[...]
    o_ref[...] = xn
    a_ref[...] = _dot(xn, w1a_n[...])
    b_ref[...] = _dot(xn, w1b_n[...])


def _proj2_body(x_ref, w1a, w1b, a_ref, b_ref):
    x = x_ref[...]
    a_ref[...] = _dot(x, w1a[...])
    b_ref[...] = _dot(x, w1b[...])


def _style_body(d_ref, w1, b1, w2, b2, o_ref):
    h = _dot(d_ref[...], w1[...]) + b1[...]
    h = 0.5 * h * (1.0 + lax.erf(h / jnp.sqrt(2.0).astype(jnp.float32)))
    o_ref[...] = _dot(h, w2[...]) + b2[...]


def _dec_body(x_ref, w, b, o_ref):
    o_ref[...] = _dot(x_ref[...], w[...]) + b[...]


def _full_spec(arr):
    return pl.BlockSpec(arr.shape, lambda i: (0,) * arr.ndim)


def _row_spec(tile, k):
    return pl.BlockSpec((tile, k), lambda i: (i, 0))


def _mlp3(x, p, tile):
    rows, k = x.shape
    out_dim = p["W3"].shape[1]
    ws = [p["W1"], p["b1"].reshape(1, -1), p["g1"].reshape(1, -1),
          p["be1"].reshape(1, -1), p["W2"], p["b2"].reshape(1, -1),
          p["g2"].reshape(1, -1), p["be2"].reshape(1, -1), p["W3"],
          p["b3"].reshape(1, -1)]
    return pl.pallas_call(
        _mlp3_body,
        grid=(rows // tile,),
        in_specs=[_row_spec(tile, k)] + [_full_spec(w) for w in ws],
        out_specs=_row_spec(tile, out_dim),
        out_shape=jax.ShapeDtypeStruct((rows, out_dim), jnp.float32),
    )(x, *ws)


def _edge_mlp(ar, bc, e, p):
    rows = ar.shape[0]
    w1c = p["W1"][2 * D:]
    ws = [w1c, p["b1"].reshape(1, -1), p["g1"].reshape(1, -1),
          p["be1"].reshape(1, -1), p["W2"], p["b2"].reshape(1, -1),
          p["g2"].reshape(1, -1), p["be2"].reshape(1, -1), p["W3"],
          p["b3"].reshape(1, -1)]
    return pl.pallas_call(
        _edge_body,
        grid=(rows // TILE_E,),
        in_specs=[_row_spec(TILE_E, D)] * 3 + [_full_spec(w) for w in ws],
        out_specs=_row_spec(TILE_E, D),
        out_shape=jax.ShapeDtypeStruct((rows, D), jnp.float32),
    )(ar, bc, e, *ws)


def _node_mlp(x, msgp, p, w1a_n, w1b_n):
    rows = x.shape[0]
    w1x, w1m = p["W1"][:D], p["W1"][D:]
    ws = [w1x, w1m, p["b1"].reshape(1, -1), p["g1"].reshape(1, -1),
          p["be1"].reshape(1, -1), p["W2"], p["b2"].reshape(1, -1),
          p["g2"].reshape(1, -1), p["be2"].reshape(1, -1), p["W3"],
          p["b3"].reshape(1, -1), w1a_n, w1b_n]
    return pl.pallas_call(
        _node_body,
        grid=(rows // TILE_N,),
        in_specs=[_row_spec(TILE_N, D),
                  pl.BlockSpec((2, TILE_N, D), lambda i: (0, i, 0))]
                 + [_full_spec(w) for w in ws],
        out_specs=[_row_spec(TILE_N, D)] * 3,
        out_shape=[jax.ShapeDtypeStruct((rows, D), jnp.float32)] * 3,
    )(x, msgp, *ws)


def _proj2(x, w1a, w1b):
    rows = x.shape[0]
    return pl.pallas_call(
        _proj2_body,
        grid=(rows // TILE_N,),
        in_specs=[_row_spec(TILE_N, D), _full_spec(w1a), _full_spec(w1b)],
        out_specs=[_row_spec(TILE_N, D)] * 2,
        out_shape=[jax.ShapeDtypeStruct((rows, D), jnp.float32)] * 2,
    )(x, w1a, w1b)


def _style_proj(dino, w1, b1, w2, b2):
    ws = [w1, b1.reshape(1, -1), w2, b2.reshape(1, -1)]
    return pl.pallas_call(
        _style_body,
        grid=(1,),
        in_specs=[_full_spec(dino)] + [_full_spec(w) for w in ws],
        out_specs=pl.BlockSpec((dino.shape[0], D), lambda i: (0, 0)),
        out_shape=jax.ShapeDtypeStruct((dino.shape[0], D), jnp.float32),
    )(dino, *ws)


def _decode(x, w, b):
    rows = x.shape[0]
    wp = jnp.pad(w, ((0, 0), (0, D - w.shape[1])))
    bp = jnp.pad(b, (0, D - b.shape[0])).reshape(1, -1)
    out = pl.pallas_call(
        _dec_body,
        grid=(rows // TILE_N,),
        in_specs=[_row_spec(TILE_N, D), _full_spec(wp), _full_spec(bp)],
        out_specs=_row_spec(TILE_N, D),
        out_shape=jax.ShapeDtypeStruct((rows, D), jnp.float32),
    )(x, wp, bp)
    return out[:, :w.shape[1]]


def _make_gather(n, kj, kk):
    e_pad = NW * kj * CH
    chg = kk * CH
    nsup = kj // kk
    mesh = plsc.VectorSubcoreMesh(core_axis_name="c", subcore_axis_name="s")

    @functools.partial(
        pl.kernel, mesh=mesh,
        out_type=[jax.ShapeDtypeStruct((e_pad, D), jnp.float32),
                  jax.ShapeDtypeStruct((e_pad, D), jnp.float32)],
        scratch_types=[
            pltpu.VMEM((kj * CH,), jnp.int32),
            pltpu.VMEM((kj * CH,), jnp.int32),
            pltpu.VMEM((chg, D), jnp.float32),
            pltpu.VMEM((chg, D), jnp.float32),
            pltpu.SemaphoreType.DMA,
            pltpu.SemaphoreType.DMA,
        ],
    )
    def gather2(a_hbm, b_hbm, row_hbm, col_hbm, ar_hbm, bc_hbm,
                ridx, cidx, rbuf, cbuf, rsem, csem):
        c = lax.axis_index("c")
        s = lax.axis_index("s")
        wid = s * 2 + c
        pltpu.sync_copy(row_hbm.at[wid], ridx)
        pltpu.sync_copy(col_hbm.at[wid], cidx)
        base = wid * kj * CH

        def body(t, carry):
            j = t * chg
            pltpu.async_copy(a_hbm.at[ridx.at[pl.ds(j, chg)]], rbuf,
                             rsem).wait()
            pltpu.sync_copy(rbuf, ar_hbm.at[pl.ds(base + j, chg)])
            pltpu.async_copy(b_hbm.at[cidx.at[pl.ds(j, chg)]], cbuf,
                             csem).wait()
            pltpu.sync_copy(cbuf, bc_hbm.at[pl.ds(base + j, chg)])
            return carry

        lax.fori_loop(0, nsup, body, 0)

    return gather2


def _make_scatter(n_acc, kj, kk):
    mesh = plsc.VectorSubcoreMesh(core_axis_name="c", subcore_axis_name="s")
    rows_per_sub = n_acc // 16
    nsup = kj // kk

    @functools.partial(
        pl.kernel, mesh=mesh,
        out_type=jax.ShapeDtypeStruct((2, n_acc, D), jnp.float32),
        scratch_types=[
            pltpu.VMEM((kj, CH), jnp.int32),
            pltpu.VMEM((kk * CH, D), jnp.float32),
            pltpu.VMEM_SHARED((n_acc, D), jnp.float32),
        ],
    )
    def scatter_add(e_hbm, col_hbm, zeros_hbm, out_hbm, cidx, ebuf, acc):
        c = lax.axis_index("c")
        s = lax.axis_index("s")
        wid = s * 2 + c
        pltpu.sync_copy(zeros_hbm.at[pl.ds(s * rows_per_sub, rows_per_sub)],
                        acc.at[pl.ds(s * rows_per_sub, rows_per_sub)])
        plsc.subcore_barrier()
        pltpu.sync_copy(col_hbm.at[wid], cidx)
        base = wid * kj * CH

        def body(t, carry):
            j = t * kk
            pltpu.sync_copy(e_hbm.at[pl.ds(base + j * CH, kk * CH)], ebuf)
            for q in range(kk):
                pltpu.sync_copy(ebuf.at[pl.ds(q * CH, CH)],
                                acc.at[cidx.at[j + q]], add=True)
            return carry

        lax.fori_loop(0, nsup, body, 0)
        plsc.subcore_barrier()
        pltpu.sync_copy(acc.at[pl.ds(s * rows_per_sub, rows_per_sub)],
                        out_hbm.at[c, pl.ds(s * rows_per_sub, rows_per_sub)])

    return scatter_add


def kernel(params, dino_feat, pos, smpl, mat, edge_attr, edge_index, batch):
    n = pos.shape[0]
    e_cnt = edge_attr.shape[0]
    kk = 2
    kj = -(-e_cnt // (NW * CH))
    kj = -(-kj // kk) * kk
    e_pad = NW * kj * CH
    n_acc = -(-(n + 1) // 128) * 128

    style = _style_proj(dino_feat, params["proj_W1"], params["proj_b1"],
                        params["proj_W2"], params["proj_b2"])

    x_in = jnp.concatenate(
        [pos, style[batch], smpl[batch], mat[batch]], axis=-1)
    k_in = x_in.shape[1]
    k_pad = -(-k_in // 8) * 8
    x_in = jnp.pad(x_in, ((0, 0), (0, k_pad - k_in)))
    ne = dict(params["node_enc"])
    ne["W1"] = jnp.pad(ne["W1"], ((0, k_pad - k_in), (0, 0)))
    x = _mlp3(x_in, ne, TILE_N)

    ea_pad = jnp.pad(edge_attr, ((0, e_pad - e_cnt), (0, 0)))
    e = _mlp3(ea_pad, params["edge_enc"], TILE_E)

    row = edge_index[0]
    col = edge_index[1]
    pad = e_pad - e_cnt
    row_g = jnp.pad(row, (0, pad)).reshape(NW, kj * CH)
    col_g = jnp.pad(col, (0, pad)).reshape(NW, kj * CH)
    col_s = jnp.pad(col, (0, pad), constant_values=n).reshape(NW, kj, CH)
    zeros_acc = jnp.zeros((n_acc, D), jnp.float32)

    gather2 = _make_gather(n, kj, kk)
    scatter_add = _make_scatter(n_acc, kj, kk)

    blocks = params["blocks"]
    a16, b16 = _proj2(x, blocks[0]["edge_mlp"]["W1"][:D],
                      blocks[0]["edge_mlp"]["W1"][D:2 * D])

    for i, blk in enumerate(blocks):
        ar, bc = gather2(a16, b16, row_g, col_g)
        e = _edge_mlp(ar, bc, e, blk["edge_mlp"])
        msgp = scatter_add(e, col_s, zeros_acc)
        nxt = blocks[i + 1]["edge_mlp"]["W1"] if i + 1 < len(blocks) \
            else blk["edge_mlp"]["W1"]
        x, a16, b16 = _node_mlp(x, msgp, blk["node_mlp"],
                                nxt[:D], nxt[D:2 * D])

    return _decode(x, params["dec_W"], params["dec_b"])


# speedup vs baseline: 1.0687x; 1.0687x over previous
import functools

import jax
import jax.numpy as jnp
from jax import lax
from jax.experimental import pallas as pl
from jax.experimental.pallas import tpu as pltpu
from jax.experimental.pallas import tpu_sc as plsc

D = 128
NW = 32
CH = 128
TILE_E = 2048
TILE_N = 2000


def _ln_relu(h, g, b):
    m = jnp.mean(h, axis=-1, keepdims=True)
    v = jnp.mean(jnp.square(h - m), axis=-1, keepdims=True)
    return jnp.maximum((h - m) * lax.rsqrt(v + 1e-5) * g + b, 0.0)


def _dot(a, w):
    return jnp.dot(a, w, preferred_element_type=jnp.float32)


def _mlp3_body(x_ref, w1, b1, g1, be1, w2, b2, g2, be2, w3, b3, o_ref):
    h = _dot(x_ref[...], w1[...]) + b1[...]
    h = _ln_relu(h, g1[...], be1[...])
    h = _dot(h, w2[...]) + b2[...]
    h = _ln_relu(h, g2[...], be2[...])
    o_ref[...] = _dot(h, w3[...]) + b3[...]


def _edge_body(ar_ref, bc_ref, e_ref, w1c, b1, g1, be1,
               w2, b2, g2, be2, w3, b3, o_ref):
    e = e_ref[...]
    h = ar_ref[...] + bc_ref[...] + _dot(e, w1c[...]) + b1[...]
    h = _ln_relu(h, g1[...], be1[...])
    h = _dot(h, w2[...]) + b2[...]
    h = _ln_relu(h, g2[...], be2[...])
    o_ref[...] = e + _dot(h, w3[...]) + b3[...]


def _node_body(x_ref, m_ref, w1x, w1m, b1, g1, be1,
               w2, b2, g2, be2, w3, b3, w1a_n, w1b_n,
               o_ref, a_ref, b_ref):
    x = x_ref[...]
    msg = m_ref[0] + m_ref[1]
    h = _dot(x, w1x[...]) + _dot(msg, w1m[...]) + b1[...]
    h = _ln_relu(h, g1[...], be1[...])
    h = _dot(h, w2[...]) + b2[...]
    h = _ln_relu(h, g2[...], be2[...])
    xn = x + _dot(h, w3[...]) + b3[...]
    o_ref[...] = xn
    a_ref[...] = _dot(xn, w1a_n[...])
    b_ref[...] = _dot(xn, w1b_n[...])


def _proj2_body(x_ref, w1a, w1b, a_ref, b_ref):
    x = x_ref[...]
    a_ref[...] = _dot(x, w1a[...])
    b_ref[...] = _dot(x, w1b[...])


def _style_body(d_ref, w1, b1, w2, b2, o_ref):
    h = _dot(d_ref[...], w1[...]) + b1[...]
    h = 0.5 * h * (1.0 + lax.erf(h / jnp.sqrt(2.0).astype(jnp.float32)))
    o_ref[...] = _dot(h, w2[...]) + b2[...]


def _dec_body(x_ref, w, b, o_ref):
    o_ref[...] = _dot(x_ref[...], w[...]) + b[...]


def _full_spec(arr):
    return pl.BlockSpec(arr.shape, lambda i: (0,) * arr.ndim)


def _row_spec(tile, k):
    return pl.BlockSpec((tile, k), lambda i: (i, 0))


def _mlp3(x, p, tile):
    rows, k = x.shape
    out_dim = p["W3"].shape[1]
    ws = [p["W1"], p["b1"].reshape(1, -1), p["g1"].reshape(1, -1),
          p["be1"].reshape(1, -1), p["W2"], p["b2"].reshape(1, -1),
          p["g2"].reshape(1, -1), p["be2"].reshape(1, -1), p["W3"],
          p["b3"].reshape(1, -1)]
    return pl.pallas_call(
        _mlp3_body,
        grid=(rows // tile,),
        in_specs=[_row_spec(tile, k)] + [_full_spec(w) for w in ws],
        out_specs=_row_spec(tile, out_dim),
        out_shape=jax.ShapeDtypeStruct((rows, out_dim), jnp.float32),
    )(x, *ws)


def _edge_mlp(ar, bc, e, p):
    rows = ar.shape[0]
    w1c = p["W1"][2 * D:]
    ws = [w1c, p["b1"].reshape(1, -1), p["g1"].reshape(1, -1),
          p["be1"].reshape(1, -1), p["W2"], p["b2"].reshape(1, -1),
          p["g2"].reshape(1, -1), p["be2"].reshape(1, -1), p["W3"],
          p["b3"].reshape(1, -1)]
    return pl.pallas_call(
        _edge_body,
        grid=(rows // TILE_E,),
        in_specs=[_row_spec(TILE_E, D)] * 3 + [_full_spec(w) for w in ws],
        out_specs=_row_spec(TILE_E, D),
        out_shape=jax.ShapeDtypeStruct((rows, D), jnp.float32),
    )(ar, bc, e, *ws)


def _node_mlp(x, msgp, p, w1a_n, w1b_n):
    rows = x.shape[0]
    w1x, w1m = p["W1"][:D], p["W1"][D:]
    ws = [w1x, w1m, p["b1"].reshape(1, -1), p["g1"].reshape(1, -1),
          p["be1"].reshape(1, -1), p["W2"], p["b2"].reshape(1, -1),
          p["g2"].reshape(1, -1), p["be2"].reshape(1, -1), p["W3"],
          p["b3"].reshape(1, -1), w1a_n, w1b_n]
    return pl.pallas_call(
        _node_body,
        grid=(rows // TILE_N,),
        in_specs=[_row_spec(TILE_N, D),
                  pl.BlockSpec((2, TILE_N, D), lambda i: (0, i, 0))]
                 + [_full_spec(w) for w in ws],
        out_specs=[_row_spec(TILE_N, D)] * 3,
        out_shape=[jax.ShapeDtypeStruct((rows, D), jnp.float32)] * 3,
    )(x, msgp, *ws)


def _proj2(x, w1a, w1b):
    rows = x.shape[0]
    return pl.pallas_call(
        _proj2_body,
        grid=(rows // TILE_N,),
        in_specs=[_row_spec(TILE_N, D), _full_spec(w1a), _full_spec(w1b)],
        out_specs=[_row_spec(TILE_N, D)] * 2,
        out_shape=[jax.ShapeDtypeStruct((rows, D), jnp.float32)] * 2,
    )(x, w1a, w1b)


def _style_proj(dino, w1, b1, w2, b2):
    ws = [w1, b1.reshape(1, -1), w2, b2.reshape(1, -1)]
    return pl.pallas_call(
        _style_body,
        grid=(1,),
        in_specs=[_full_spec(dino)] + [_full_spec(w) for w in ws],
        out_specs=pl.BlockSpec((dino.shape[0], D), lambda i: (0, 0)),
        out_shape=jax.ShapeDtypeStruct((dino.shape[0], D), jnp.float32),
    )(dino, *ws)


def _decode(x, w, b):
    rows = x.shape[0]
    wp = jnp.pad(w, ((0, 0), (0, D - w.shape[1])))
    bp = jnp.pad(b, (0, D - b.shape[0])).reshape(1, -1)
    out = pl.pallas_call(
        _dec_body,
        grid=(rows // TILE_N,),
        in_specs=[_row_spec(TILE_N, D), _full_spec(wp), _full_spec(bp)],
        out_specs=_row_spec(TILE_N, D),
        out_shape=jax.ShapeDtypeStruct((rows, D), jnp.float32),
    )(x, wp, bp)
    return out[:, :w.shape[1]]


def _make_gather(n, kj):
    e_pad = NW * kj * CH
    mesh = plsc.VectorSubcoreMesh(core_axis_name="c", subcore_axis_name="s")

    @functools.partial(
        pl.kernel, mesh=mesh,
        out_type=[jax.ShapeDtypeStruct((e_pad, D), jnp.float32),
                  jax.ShapeDtypeStruct((e_pad, D), jnp.float32)],
        scratch_types=[
            pltpu.VMEM((kj, CH), jnp.int32),
            pltpu.VMEM((kj, CH), jnp.int32),
            pltpu.VMEM((CH, D), jnp.float32),
            pltpu.VMEM((CH, D), jnp.float32),
            pltpu.SemaphoreType.DMA,
            pltpu.SemaphoreType.DMA,
        ],
    )
    def gather2(a_hbm, b_hbm, row_hbm, col_hbm, ar_hbm, bc_hbm,
                ridx, cidx, rbuf, cbuf, rsem, csem):
        c = lax.axis_index("c")
        s = lax.axis_index("s")
        wid = s * 2 + c
        pltpu.sync_copy(row_hbm.at[wid], ridx)
        pltpu.sync_copy(col_hbm.at[wid], cidx)
        base = wid * kj * CH

        def body(j, carry):
            pltpu.async_copy(a_hbm.at[ridx.at[j]], rbuf, rsem).wait()
            pltpu.sync_copy(rbuf, ar_hbm.at[pl.ds(base + j * CH, CH)])
            pltpu.async_copy(b_hbm.at[cidx.at[j]], cbuf, csem).wait()
            pltpu.sync_copy(cbuf, bc_hbm.at[pl.ds(base + j * CH, CH)])
            return carry

        lax.fori_loop(0, kj, body, 0)

    return gather2


def _make_scatter(n_acc, kj):
    mesh = plsc.VectorSubcoreMesh(core_axis_name="c", subcore_axis_name="s")
    rows_per_sub = n_acc // 16

    @functools.partial(
        pl.kernel, mesh=mesh,
        out_type=jax.ShapeDtypeStruct((2, n_acc, D), jnp.float32),
        scratch_types=[
            pltpu.VMEM((kj, CH), jnp.int32),
            pltpu.VMEM((CH, D), jnp.float32),
            pltpu.VMEM_SHARED((n_acc, D), jnp.float32),
        ],
    )
    def scatter_add(e_hbm, col_hbm, zeros_hbm, out_hbm, cidx, ebuf, acc):
        c = lax.axis_index("c")
        s = lax.axis_index("s")
        wid = s * 2 + c
        pltpu.sync_copy(zeros_hbm.at[pl.ds(s * rows_per_sub, rows_per_sub)],
                        acc.at[pl.ds(s * rows_per_sub, rows_per_sub)])
        plsc.subcore_barrier()
        pltpu.sync_copy(col_hbm.at[wid], cidx)
        base = wid * kj * CH

        def body(j, carry):
            pltpu.sync_copy(e_hbm.at[pl.ds(base + j * CH, CH)], ebuf)
            pltpu.sync_copy(ebuf, acc.at[cidx.at[j]], add=True)
            return carry

        lax.fori_loop(0, kj, body, 0)
        plsc.subcore_barrier()
        pltpu.sync_copy(acc.at[pl.ds(s * rows_per_sub, rows_per_sub)],
                        out_hbm.at[c, pl.ds(s * rows_per_sub, rows_per_sub)])

    return scatter_add


def kernel(params, dino_feat, pos, smpl, mat, edge_attr, edge_index, batch):
    n = pos.shape[0]
    e_cnt = edge_attr.shape[0]
    kj = -(-e_cnt // (NW * CH))
    e_pad = NW * kj * CH
    n_acc = -(-(n + 1) // 128) * 128

    style = _style_proj(dino_feat, params["proj_W1"], params["proj_b1"],
                        params["proj_W2"], params["proj_b2"])

    x_in = jnp.concatenate(
        [pos, style[batch], smpl[batch], mat[batch]], axis=-1)
    k_in = x_in.shape[1]
    k_pad = -(-k_in // 8) * 8
    x_in = jnp.pad(x_in, ((0, 0), (0, k_pad - k_in)))
    ne = dict(params["node_enc"])
    ne["W1"] = jnp.pad(ne["W1"], ((0, k_pad - k_in), (0, 0)))
    x = _mlp3(x_in, ne, TILE_N)

    row = edge_index[0]
    col = edge_index[1]
    pad = e_pad - e_cnt
    perm = jnp.argsort(col)
    row_p = row[perm]
    col_p = col[perm]
    ea_p = edge_attr[perm]
    row_g = jnp.pad(row_p, (0, pad)).reshape(NW, kj, CH)
    col_g = jnp.pad(col_p, (0, pad)).reshape(NW, kj, CH)
    col_s = jnp.pad(col_p, (0, pad), constant_values=n).reshape(NW, kj, CH)
    e = _mlp3(jnp.pad(ea_p, ((0, pad), (0, 0))), params["edge_enc"], TILE_E)
    zeros_acc = jnp.zeros((n_acc, D), jnp.float32)

    gather2 = _make_gather(n, kj)
    scatter_add = _make_scatter(n_acc, kj)

    blocks = params["blocks"]
    a16, b16 = _proj2(x, blocks[0]["edge_mlp"]["W1"][:D],
                      blocks[0]["edge_mlp"]["W1"][D:2 * D])

    for i, blk in enumerate(blocks):
        ar, bc = gather2(a16, b16, row_g, col_g)
        e = _edge_mlp(ar, bc, e, blk["edge_mlp"])
        msgp = scatter_add(e, col_s, zeros_acc)
        nxt = blocks[i + 1]["edge_mlp"]["W1"] if i + 1 < len(blocks) \
            else blk["edge_mlp"]["W1"]
        x, a16, b16 = _node_mlp(x, msgp, blk["node_mlp"],
                                nxt[:D], nxt[D:2 * D])

    return _decode(x, params["dec_W"], params["dec_b"])


# speedup vs baseline: 1.3375x; 1.2515x over previous
import functools

import jax
import jax.numpy as jnp
from jax import lax
from jax.experimental import pallas as pl
from jax.experimental.pallas import tpu as pltpu
from jax.experimental.pallas import tpu_sc as plsc

D = 128
NW = 32
CH = 128
TILE_E = 2048
TILE_N = 2000


def _ln_relu(h, g, b):
    m = jnp.mean(h, axis=-1, keepdims=True)
    v = jnp.mean(jnp.square(h - m), axis=-1, keepdims=True)
    return jnp.maximum((h - m) * lax.rsqrt(v + 1e-5) * g + b, 0.0)


def _dot(a, w):
    return jnp.dot(a, w, preferred_element_type=jnp.float32)


def _mlp3_body(x_ref, w1, b1, g1, be1, w2, b2, g2, be2, w3, b3, o_ref):
    h = _dot(x_ref[...], w1[...]) + b1[...]
    h = _ln_relu(h, g1[...], be1[...])
    h = _dot(h, w2[...]) + b2[...]
    h = _ln_relu(h, g2[...], be2[...])
    o_ref[...] = _dot(h, w3[...]) + b3[...]


def _edge_body(ar_ref, bc_ref, e_ref, w1c, b1, g1, be1,
               w2, b2, g2, be2, w3, b3, o_ref):
    e = e_ref[...]
    h = ar_ref[...] + bc_ref[...] + _dot(e, w1c[...]) + b1[...]
    h = _ln_relu(h, g1[...], be1[...])
    h = _dot(h, w2[...]) + b2[...]
    h = _ln_relu(h, g2[...], be2[...])
    o_ref[...] = e + _dot(h, w3[...]) + b3[...]


def _node_body(x_ref, m_ref, w1x, w1m, b1, g1, be1,
               w2, b2, g2, be2, w3, b3, w1a_n, w1b_n,
               o_ref, a_ref, b_ref):
    x = x_ref[...]
    msg = m_ref[0] + m_ref[1]
    h = _dot(x, w1x[...]) + _dot(msg, w1m[...]) + b1[...]
    h = _ln_relu(h, g1[...], be1[...])
    h = _dot(h, w2[...]) + b2[...]
    h = _ln_relu(h, g2[...], be2[...])
    xn = x + _dot(h, w3[...]) + b3[...]
    o_ref[...] = xn
    a_ref[...] = _dot(xn, w1a_n[...])
    b_ref[...] = _dot(xn, w1b_n[...])


def _proj2_body(x_ref, w1a, w1b, a_ref, b_ref):
    x = x_ref[...]
    a_ref[...] = _dot(x, w1a[...])
    b_ref[...] = _dot(x, w1b[...])


def _style_body(d_ref, w1, b1, w2, b2, o_ref):
    h = _dot(d_ref[...], w1[...]) + b1[...]
    h = 0.5 * h * (1.0 + lax.erf(h / jnp.sqrt(2.0).astype(jnp.float32)))
    o_ref[...] = _dot(h, w2[...]) + b2[...]


def _dec_body(x_ref, w, b, o_ref):
    o_ref[...] = _dot(x_ref[...], w[...]) + b[...]


def _full_spec(arr):
    return pl.BlockSpec(arr.shape, lambda i: (0,) * arr.ndim)


def _row_spec(tile, k):
    return pl.BlockSpec((tile, k), lambda i: (i, 0))


def _mlp3(x, p, tile):
    rows, k = x.shape
    out_dim = p["W3"].shape[1]
    ws = [p["W1"], p["b1"].reshape(1, -1), p["g1"].reshape(1, -1),
          p["be1"].reshape(1, -1), p["W2"], p["b2"].reshape(1, -1),
          p["g2"].reshape(1, -1), p["be2"].reshape(1, -1), p["W3"],
          p["b3"].reshape(1, -1)]
    return pl.pallas_call(
        _mlp3_body,
        grid=(rows // tile,),
        in_specs=[_row_spec(tile, k)] + [_full_spec(w) for w in ws],
        out_specs=_row_spec(tile, out_dim),
        out_shape=jax.ShapeDtypeStruct((rows, out_dim), jnp.float32),
    )(x, *ws)


def _edge_mlp(ar, bc, e, p):
    rows = ar.shape[0]
    w1c = p["W1"][2 * D:]
    ws = [w1c, p["b1"].reshape(1, -1), p["g1"].reshape(1, -1),
          p["be1"].reshape(1, -1), p["W2"], p["b2"].reshape(1, -1),
          p["g2"].reshape(1, -1), p["be2"].reshape(1, -1), p["W3"],
          p["b3"].reshape(1, -1)]
    return pl.pallas_call(
        _edge_body,
        grid=(rows // TILE_E,),
        in_specs=[_row_spec(TILE_E, D)] * 3 + [_full_spec(w) for w in ws],
        out_specs=_row_spec(TILE_E, D),
        out_shape=jax.ShapeDtypeStruct((rows, D), jnp.float32),
    )(ar, bc, e, *ws)


def _node_mlp(x, msgp, p, w1a_n, w1b_n):
    rows = x.shape[0]
    w1x, w1m = p["W1"][:D], p["W1"][D:]
    ws = [w1x, w1m, p["b1"].reshape(1, -1), p["g1"].reshape(1, -1),
          p["be1"].reshape(1, -1), p["W2"], p["b2"].reshape(1, -1),
          p["g2"].reshape(1, -1), p["be2"].reshape(1, -1), p["W3"],
          p["b3"].reshape(1, -1), w1a_n, w1b_n]
    return pl.pallas_call(
        _node_body,
        grid=(rows // TILE_N,),
        in_specs=[_row_spec(TILE_N, D),
                  pl.BlockSpec((2, TILE_N, D), lambda i: (0, i, 0))]
                 + [_full_spec(w) for w in ws],
        out_specs=[_row_spec(TILE_N, D)] * 3,
        out_shape=[jax.ShapeDtypeStruct((rows, D), jnp.float32)] * 3,
    )(x, msgp, *ws)


def _proj2(x, w1a, w1b):
    rows = x.shape[0]
    return pl.pallas_call(
        _proj2_body,
        grid=(rows // TILE_N,),
        in_specs=[_row_spec(TILE_N, D), _full_spec(w1a), _full_spec(w1b)],
        out_specs=[_row_spec(TILE_N, D)] * 2,
        out_shape=[jax.ShapeDtypeStruct((rows, D), jnp.float32)] * 2,
    )(x, w1a, w1b)


def _style_proj(dino, w1, b1, w2, b2):
    ws = [w1, b1.reshape(1, -1), w2, b2.reshape(1, -1)]
    return pl.pallas_call(
        _style_body,
        grid=(1,),
        in_specs=[_full_spec(dino)] + [_full_spec(w) for w in ws],
        out_specs=pl.BlockSpec((dino.shape[0], D), lambda i: (0, 0)),
        out_shape=jax.ShapeDtypeStruct((dino.shape[0], D), jnp.float32),
    )(dino, *ws)


def _decode(x, w, b):
    rows = x.shape[0]
    wp = jnp.pad(w, ((0, 0), (0, D - w.shape[1])))
    bp = jnp.pad(b, (0, D - b.shape[0])).reshape(1, -1)
    out = pl.pallas_call(
        _dec_body,
        grid=(rows // TILE_N,),
        in_specs=[_row_spec(TILE_N, D), _full_spec(wp), _full_spec(bp)],
        out_specs=_row_spec(TILE_N, D),
        out_shape=jax.ShapeDtypeStruct((rows, D), jnp.float32),
    )(x, wp, bp)
    return out[:, :w.shape[1]]


def _make_gather(n, kj):
    e_pad = NW * kj * CH
    mesh = plsc.VectorSubcoreMesh(core_axis_name="c", subcore_axis_name="s")

    @functools.partial(
        pl.kernel, mesh=mesh,
        out_type=[jax.ShapeDtypeStruct((e_pad, D), jnp.float32),
                  jax.ShapeDtypeStruct((e_pad, D), jnp.float32)],
        scratch_types=[
            pltpu.VMEM((kj, CH), jnp.int32),
            pltpu.VMEM((kj, CH), jnp.int32),
            pltpu.VMEM((CH, D), jnp.float32),
            pltpu.VMEM((CH, D), jnp.float32),
            pltpu.SemaphoreType.DMA,
            pltpu.SemaphoreType.DMA,
        ],
    )
    def gather2(a_hbm, b_hbm, row_hbm, col_hbm, ar_hbm, bc_hbm,
                ridx, cidx, rbuf, cbuf, rsem, csem):
        c = lax.axis_index("c")
        s = lax.axis_index("s")
        wid = s * 2 + c
        pltpu.sync_copy(row_hbm.at[wid], ridx)
        pltpu.sync_copy(col_hbm.at[wid], cidx)
        base = wid * kj * CH

        def body(j, carry):
            cr = pltpu.async_copy(a_hbm.at[ridx.at[j]], rbuf, rsem)
            cc = pltpu.async_copy(b_hbm.at[cidx.at[j]], cbuf, csem)
            cr.wait()
            pltpu.sync_copy(rbuf, ar_hbm.at[pl.ds(base + j * CH, CH)])
            cc.wait()
            pltpu.sync_copy(cbuf, bc_hbm.at[pl.ds(base + j * CH, CH)])
            return carry

        lax.fori_loop(0, kj, body, 0)

    return gather2


def _make_scatter(n_acc, kj):
    mesh = plsc.VectorSubcoreMesh(core_axis_name="c", subcore_axis_name="s")
    rows_per_sub = n_acc // 16

    @functools.partial(
        pl.kernel, mesh=mesh,
        out_type=jax.ShapeDtypeStruct((2, n_acc, D), jnp.float32),
        scratch_types=[
            pltpu.VMEM((kj, CH), jnp.int32),
            pltpu.VMEM((CH, D), jnp.float32),
            pltpu.VMEM((CH, D), jnp.float32),
            pltpu.VMEM_SHARED((n_acc, D), jnp.float32),
            pltpu.SemaphoreType.DMA,
            pltpu.SemaphoreType.DMA,
        ],
    )
    def scatter_add(e_hbm, col_hbm, zeros_hbm, out_hbm,
                    cidx, eb0, eb1, acc, es0, es1):
        c = lax.axis_index("c")
        s = lax.axis_index("s")
        wid = s * 2 + c
        pltpu.sync_copy(zeros_hbm.at[pl.ds(s * rows_per_sub, rows_per_sub)],
                        acc.at[pl.ds(s * rows_per_sub, rows_per_sub)])
        plsc.subcore_barrier()
        pltpu.sync_copy(col_hbm.at[wid], cidx)
        base = wid * kj * CH

        def body(t, carry):
            j = t * 2
            c0 = pltpu.async_copy(e_hbm.at[pl.ds(base + j * CH, CH)],
                                  eb0, es0)
            c1 = pltpu.async_copy(e_hbm.at[pl.ds(base + (j + 1) * CH, CH)],
                                  eb1, es1)
            c0.wait()
            pltpu.sync_copy(eb0, acc.at[cidx.at[j]], add=True)
            c1.wait()
            pltpu.sync_copy(eb1, acc.at[cidx.at[j + 1]], add=True)
            return carry

        lax.fori_loop(0, kj // 2, body, 0)
        plsc.subcore_barrier()
        pltpu.sync_copy(acc.at[pl.ds(s * rows_per_sub, rows_per_sub)],
                        out_hbm.at[c, pl.ds(s * rows_per_sub, rows_per_sub)])

    return scatter_add


def kernel(params, dino_feat, pos, smpl, mat, edge_attr, edge_index, batch):
    n = pos.shape[0]
    e_cnt = edge_attr.shape[0]
    kj = -(-e_cnt // (NW * CH))
    kj += kj % 2
    e_pad = NW * kj * CH
    n_acc = -(-(n + 1) // 128) * 128

    style = _style_proj(dino_feat, params["proj_W1"], params["proj_b1"],
                        params["proj_W2"], params["proj_b2"])

    x_in = jnp.concatenate(
        [pos, style[batch], smpl[batch], mat[batch]], axis=-1)
    k_in = x_in.shape[1]
    k_pad = -(-k_in // 8) * 8
    x_in = jnp.pad(x_in, ((0, 0), (0, k_pad - k_in)))
    ne = dict(params["node_enc"])
    ne["W1"] = jnp.pad(ne["W1"], ((0, k_pad - k_in), (0, 0)))
    x = _mlp3(x_in, ne, TILE_N)

    row = edge_index[0]
    col = edge_index[1]
    pad = e_pad - e_cnt
    row_p = row
    col_p = col
    ea_p = edge_attr
    row_g = jnp.pad(row_p, (0, pad)).reshape(NW, kj, CH)
    col_g = jnp.pad(col_p, (0, pad)).reshape(NW, kj, CH)
    col_s = jnp.pad(col_p, (0, pad), constant_values=n).reshape(NW, kj, CH)
    e = _mlp3(jnp.pad(ea_p, ((0, pad), (0, 0))), params["edge_enc"], TILE_E)
    zeros_acc = jnp.zeros((n_acc, D), jnp.float32)

    gather2 = _make_gather(n, kj)
    scatter_add = _make_scatter(n_acc, kj)

    blocks = params["blocks"]
    a16, b16 = _proj2(x, blocks[0]["edge_mlp"]["W1"][:D],
                      blocks[0]["edge_mlp"]["W1"][D:2 * D])

    for i, blk in enumerate(blocks):
        ar, bc = gather2(a16, b16, row_g, col_g)
        e = _edge_mlp(ar, bc, e, blk["edge_mlp"])
        msgp = scatter_add(e, col_s, zeros_acc)
        nxt = blocks[i + 1]["edge_mlp"]["W1"] if i + 1 < len(blocks) \
            else blk["edge_mlp"]["W1"]
        x, a16, b16 = _node_mlp(x, msgp, blk["node_mlp"],
                                nxt[:D], nxt[D:2 * D])

    return _decode(x, params["dec_W"], params["dec_b"])


# speedup vs baseline: 1.3684x; 1.0230x over previous
import functools

import jax
import jax.numpy as jnp
from jax import lax
from jax.experimental import pallas as pl
from jax.experimental.pallas import tpu as pltpu
from jax.experimental.pallas import tpu_sc as plsc

D = 128
NW = 32
CH = 128
TILE_E = 2048
TILE_N = 2000


def _ln_relu(h, g, b):
    m = jnp.mean(h, axis=-1, keepdims=True)
    v = jnp.mean(jnp.square(h - m), axis=-1, keepdims=True)
    return jnp.maximum((h - m) * lax.rsqrt(v + 1e-5) * g + b, 0.0)


def _dot(a, w):
    return jnp.dot(a, w, preferred_element_type=jnp.float32)


def _mlp3_body(x_ref, w1, b1, g1, be1, w2, b2, g2, be2, w3, b3, o_ref):
    h = _dot(x_ref[...], w1[...]) + b1[...]
    h = _ln_relu(h, g1[...], be1[...])
    h = _dot(h, w2[...]) + b2[...]
    h = _ln_relu(h, g2[...], be2[...])
    o_ref[...] = _dot(h, w3[...]) + b3[...]


def _edge_body(ar_ref, bc_ref, e_ref, w1c, b1, g1, be1,
               w2, b2, g2, be2, w3, b3, o_ref):
    e = e_ref[...]
    h = ar_ref[...] + bc_ref[...] + _dot(e, w1c[...]) + b1[...]
    h = _ln_relu(h, g1[...], be1[...])
    h = _dot(h, w2[...]) + b2[...]
    h = _ln_relu(h, g2[...], be2[...])
    o_ref[...] = e + _dot(h, w3[...]) + b3[...]


def _node_body(x_ref, m_ref, w1x, w1m, b1, g1, be1,
               w2, b2, g2, be2, w3, b3, w1a_n, w1b_n,
               o_ref, a_ref, b_ref):
    x = x_ref[...]
    msg = m_ref[0] + m_ref[1]
    h = _dot(x, w1x[...]) + _dot(msg, w1m[...]) + b1[...]
    h = _ln_relu(h, g1[...], be1[...])
    h = _dot(h, w2[...]) + b2[...]
    h = _ln_relu(h, g2[...], be2[...])
    xn = x + _dot(h, w3[...]) + b3[...]
    o_ref[...] = xn
    a_ref[...] = _dot(xn, w1a_n[...])
    b_ref[...] = _dot(xn, w1b_n[...])


def _proj2_body(x_ref, w1a, w1b, a_ref, b_ref):
    x = x_ref[...]
    a_ref[...] = _dot(x, w1a[...])
    b_ref[...] = _dot(x, w1b[...])


def _style_body(d_ref, w1, b1, w2, b2, o_ref):
    h = _dot(d_ref[...], w1[...]) + b1[...]
    h = 0.5 * h * (1.0 + lax.erf(h / jnp.sqrt(2.0).astype(jnp.float32)))
    o_ref[...] = _dot(h, w2[...]) + b2[...]


def _dec_body(x_ref, w, b, o_ref):
    o_ref[...] = _dot(x_ref[...], w[...]) + b[...]


def _full_spec(arr):
    return pl.BlockSpec(arr.shape, lambda i: (0,) * arr.ndim)


def _row_spec(tile, k):
    return pl.BlockSpec((tile, k), lambda i: (i, 0))


def _mlp3(x, p, tile):
    rows, k = x.shape
    out_dim = p["W3"].shape[1]
    ws = [p["W1"], p["b1"].reshape(1, -1), p["g1"].reshape(1, -1),
          p["be1"].reshape(1, -1), p["W2"], p["b2"].reshape(1, -1),
          p["g2"].reshape(1, -1), p["be2"].reshape(1, -1), p["W3"],
          p["b3"].reshape(1, -1)]
    return pl.pallas_call(
        _mlp3_body,
        grid=(rows // tile,),
        in_specs=[_row_spec(tile, k)] + [_full_spec(w) for w in ws],
        out_specs=_row_spec(tile, out_dim),
        out_shape=jax.ShapeDtypeStruct((rows, out_dim), jnp.float32),
    )(x, *ws)


def _edge_mlp(ar, bc, e, p):
    rows = ar.shape[0]
    w1c = p["W1"][2 * D:]
    ws = [w1c, p["b1"].reshape(1, -1), p["g1"].reshape(1, -1),
          p["be1"].reshape(1, -1), p["W2"], p["b2"].reshape(1, -1),
          p["g2"].reshape(1, -1), p["be2"].reshape(1, -1), p["W3"],
          p["b3"].reshape(1, -1)]
    return pl.pallas_call(
        _edge_body,
        grid=(rows // TILE_E,),
        in_specs=[_row_spec(TILE_E, D)] * 3 + [_full_spec(w) for w in ws],
        out_specs=_row_spec(TILE_E, D),
        out_shape=jax.ShapeDtypeStruct((rows, D), jnp.float32),
    )(ar, bc, e, *ws)


def _node_mlp(x, msgp, p, w1a_n, w1b_n):
    rows = x.shape[0]
    w1x, w1m = p["W1"][:D], p["W1"][D:]
    ws = [w1x, w1m, p["b1"].reshape(1, -1), p["g1"].reshape(1, -1),
          p["be1"].reshape(1, -1), p["W2"], p["b2"].reshape(1, -1),
          p["g2"].reshape(1, -1), p["be2"].reshape(1, -1), p["W3"],
          p["b3"].reshape(1, -1), w1a_n, w1b_n]
    return pl.pallas_call(
        _node_body,
        grid=(rows // TILE_N,),
        in_specs=[_row_spec(TILE_N, D),
                  pl.BlockSpec((2, TILE_N, D), lambda i: (0, i, 0))]
                 + [_full_spec(w) for w in ws],
        out_specs=[_row_spec(TILE_N, D)] * 3,
        out_shape=[jax.ShapeDtypeStruct((rows, D), jnp.float32)] * 3,
    )(x, msgp, *ws)


def _proj2(x, w1a, w1b):
    rows = x.shape[0]
    return pl.pallas_call(
        _proj2_body,
        grid=(rows // TILE_N,),
        in_specs=[_row_spec(TILE_N, D), _full_spec(w1a), _full_spec(w1b)],
        out_specs=[_row_spec(TILE_N, D)] * 2,
        out_shape=[jax.ShapeDtypeStruct((rows, D), jnp.float32)] * 2,
    )(x, w1a, w1b)


def _style_proj(dino, w1, b1, w2, b2):
    ws = [w1, b1.reshape(1, -1), w2, b2.reshape(1, -1)]
    return pl.pallas_call(
        _style_body,
        grid=(1,),
        in_specs=[_full_spec(dino)] + [_full_spec(w) for w in ws],
        out_specs=pl.BlockSpec((dino.shape[0], D), lambda i: (0, 0)),
        out_shape=jax.ShapeDtypeStruct((dino.shape[0], D), jnp.float32),
    )(dino, *ws)


def _decode(x, w, b):
    rows = x.shape[0]
    wp = jnp.pad(w, ((0, 0), (0, D - w.shape[1])))
    bp = jnp.pad(b, (0, D - b.shape[0])).reshape(1, -1)
    out = pl.pallas_call(
        _dec_body,
        grid=(rows // TILE_N,),
        in_specs=[_row_spec(TILE_N, D), _full_spec(wp), _full_spec(bp)],
        out_specs=_row_spec(TILE_N, D),
        out_shape=jax.ShapeDtypeStruct((rows, D), jnp.float32),
    )(x, wp, bp)
    return out[:, :w.shape[1]]


def _make_gather(n, kj):
    e_pad = NW * kj * CH
    mesh = plsc.VectorSubcoreMesh(core_axis_name="c", subcore_axis_name="s")

    @functools.partial(
        pl.kernel, mesh=mesh,
        out_type=[jax.ShapeDtypeStruct((e_pad, D), jnp.float32),
                  jax.ShapeDtypeStruct((e_pad, D), jnp.float32)],
        scratch_types=[
            pltpu.VMEM((kj, CH), jnp.int32),
            pltpu.VMEM((kj, CH), jnp.int32),
            pltpu.VMEM((2 * CH, D), jnp.float32),
            pltpu.VMEM((2 * CH, D), jnp.float32),
            pltpu.SemaphoreType.DMA,
            pltpu.SemaphoreType.DMA,
        ],
    )
    def gather2(a_hbm, b_hbm, row_hbm, col_hbm, ar_hbm, bc_hbm,
                ridx, cidx, rbuf, cbuf, rsem, csem):
        c = lax.axis_index("c")
        s = lax.axis_index("s")
        wid = s * 2 + c
        pltpu.sync_copy(row_hbm.at[wid], ridx)
        pltpu.sync_copy(col_hbm.at[wid], cidx)
        base = wid * kj * CH

        def body(t, carry):
            j = t * 2
            g0 = pltpu.async_copy(a_hbm.at[ridx.at[j]],
                                  rbuf.at[pl.ds(0, CH)], rsem)
            g1 = pltpu.async_copy(a_hbm.at[ridx.at[j + 1]],
                                  rbuf.at[pl.ds(CH, CH)], rsem)
            g2 = pltpu.async_copy(b_hbm.at[cidx.at[j]],
                                  cbuf.at[pl.ds(0, CH)], csem)
            g3 = pltpu.async_copy(b_hbm.at[cidx.at[j + 1]],
                                  cbuf.at[pl.ds(CH, CH)], csem)
            g0.wait()
            g1.wait()
            pltpu.sync_copy(rbuf, ar_hbm.at[pl.ds(base + j * CH, 2 * CH)])
            g2.wait()
            g3.wait()
            pltpu.sync_copy(cbuf, bc_hbm.at[pl.ds(base + j * CH, 2 * CH)])
            return carry

        lax.fori_loop(0, kj // 2, body, 0)

    return gather2


def _make_scatter(n_acc, kj):
    mesh = plsc.VectorSubcoreMesh(core_axis_name="c", subcore_axis_name="s")
    rows_per_sub = n_acc // 16

    @functools.partial(
        pl.kernel, mesh=mesh,
        out_type=jax.ShapeDtypeStruct((2, n_acc, D), jnp.float32),
        scratch_types=[
            pltpu.VMEM((kj, CH), jnp.int32),
            pltpu.VMEM((CH, D), jnp.float32),
            pltpu.VMEM((CH, D), jnp.float32),
            pltpu.VMEM_SHARED((n_acc, D), jnp.float32),
            pltpu.SemaphoreType.DMA,
            pltpu.SemaphoreType.DMA,
            pltpu.SemaphoreType.DMA,
            pltpu.SemaphoreType.DMA,
        ],
    )
    def scatter_add(e_hbm, col_hbm, zeros_hbm, out_hbm,
                    cidx, eb0, eb1, acc, es0, es1, as0, as1):
        c = lax.axis_index("c")
        s = lax.axis_index("s")
        wid = s * 2 + c
        pltpu.sync_copy(zeros_hbm.at[pl.ds(s * rows_per_sub, rows_per_sub)],
                        acc.at[pl.ds(s * rows_per_sub, rows_per_sub)])
        plsc.subcore_barrier()
        pltpu.sync_copy(col_hbm.at[wid], cidx)
        base = wid * kj * CH

        def body(t, carry):
            j = t * 2
            c0 = pltpu.async_copy(e_hbm.at[pl.ds(base + j * CH, CH)],
                                  eb0, es0)
            c1 = pltpu.async_copy(e_hbm.at[pl.ds(base + (j + 1) * CH, CH)],
                                  eb1, es1)
            c0.wait()
            a0 = pltpu.async_copy(eb0, acc.at[cidx.at[j]], as0, add=True)
            c1.wait()
            a1 = pltpu.async_copy(eb1, acc.at[cidx.at[j + 1]], as1,
                                  add=True)
            a0.wait()
            a1.wait()
            return carry

        lax.fori_loop(0, kj // 2, body, 0)
        plsc.subcore_barrier()
        pltpu.sync_copy(acc.at[pl.ds(s * rows_per_sub, rows_per_sub)],
                        out_hbm.at[c, pl.ds(s * rows_per_sub, rows_per_sub)])

    return scatter_add


def kernel(params, dino_feat, pos, smpl, mat, edge_attr, edge_index, batch):
    n = pos.shape[0]
    e_cnt = edge_attr.shape[0]
    kj = -(-e_cnt // (NW * CH))
    kj += kj % 2
    e_pad = NW * kj * CH
    n_acc = -(-(n + 1) // 128) * 128

    style = _style_proj(dino_feat, params["proj_W1"], params["proj_b1"],
                        params["proj_W2"], params["proj_b2"])

    x_in = jnp.concatenate(
        [pos, style[batch], smpl[batch], mat[batch]], axis=-1)
    k_in = x_in.shape[1]
    k_pad = -(-k_in // 8) * 8
    x_in = jnp.pad(x_in, ((0, 0), (0, k_pad - k_in)))
    ne = dict(params["node_enc"])
    ne["W1"] = jnp.pad(ne["W1"], ((0, k_pad - k_in), (0, 0)))
    x = _mlp3(x_in, ne, TILE_N)

    row = edge_index[0]
    col = edge_index[1]
    pad = e_pad - e_cnt
    row_p = row
    col_p = col
    ea_p = edge_attr
    row_g = jnp.pad(row_p, (0, pad)).reshape(NW, kj, CH)
    col_g = jnp.pad(col_p, (0, pad)).reshape(NW, kj, CH)
    col_s = jnp.pad(col_p, (0, pad), constant_values=n).reshape(NW, kj, CH)
    e = _mlp3(jnp.pad(ea_p, ((0, pad), (0, 0))), params["edge_enc"], TILE_E)
    zeros_acc = jnp.zeros((n_acc, D), jnp.float32)

    gather2 = _make_gather(n, kj)
    scatter_add = _make_scatter(n_acc, kj)

    blocks = params["blocks"]
    a16, b16 = _proj2(x, blocks[0]["edge_mlp"]["W1"][:D],
                      blocks[0]["edge_mlp"]["W1"][D:2 * D])

    for i, blk in enumerate(blocks):
        ar, bc = gather2(a16, b16, row_g, col_g)
        e = _edge_mlp(ar, bc, e, blk["edge_mlp"])
        msgp = scatter_add(e, col_s, zeros_acc)
        nxt = blocks[i + 1]["edge_mlp"]["W1"] if i + 1 < len(blocks) \
            else blk["edge_mlp"]["W1"]
        x, a16, b16 = _node_mlp(x, msgp, blk["node_mlp"],
                                nxt[:D], nxt[D:2 * D])

    return _decode(x, params["dec_W"], params["dec_b"])


# speedup vs baseline: 1.3701x; 1.0013x over previous
import functools

import jax
import jax.numpy as jnp
from jax import lax
from jax.experimental import pallas as pl
from jax.experimental.pallas import tpu as pltpu
from jax.experimental.pallas import tpu_sc as plsc

D = 128
NW = 32
CH = 128
TILE_E = 2048
TILE_N = 2000


def _ln_relu(h, g, b):
    m = jnp.mean(h, axis=-1, keepdims=True)
    v = jnp.mean(jnp.square(h - m), axis=-1, keepdims=True)
    return jnp.maximum((h - m) * lax.rsqrt(v + 1e-5) * g + b, 0.0)


def _dot(a, w):
    return jnp.dot(a, w, preferred_element_type=jnp.float32)


def _mlp3_body(x_ref, w1, b1, g1, be1, w2, b2, g2, be2, w3, b3, o_ref):
    h = _dot(x_ref[...], w1[...]) + b1[...]
    h = _ln_relu(h, g1[...], be1[...])
    h = _dot(h, w2[...]) + b2[...]
    h = _ln_relu(h, g2[...], be2[...])
    o_ref[...] = _dot(h, w3[...]) + b3[...]


def _edge_body(ar_ref, bc_ref, e_ref, w1c, b1, g1, be1,
               w2, b2, g2, be2, w3, b3, o_ref):
    e = e_ref[...]
    h = ar_ref[...] + bc_ref[...] + _dot(e, w1c[...]) + b1[...]
    h = _ln_relu(h, g1[...], be1[...])
    h = _dot(h, w2[...]) + b2[...]
    h = _ln_relu(h, g2[...], be2[...])
    o_ref[...] = e + _dot(h, w3[...]) + b3[...]


def _node_body(x_ref, m_ref, w1x, w1m, b1, g1, be1,
               w2, b2, g2, be2, w3, b3, w1a_n, w1b_n,
               o_ref, a_ref, b_ref):
    x = x_ref[...]
    msg = m_ref[0] + m_ref[1]
    h = _dot(x, w1x[...]) + _dot(msg, w1m[...]) + b1[...]
    h = _ln_relu(h, g1[...], be1[...])
    h = _dot(h, w2[...]) + b2[...]
    h = _ln_relu(h, g2[...], be2[...])
    xn = x + _dot(h, w3[...]) + b3[...]
    o_ref[...] = xn
    a_ref[...] = _dot(xn, w1a_n[...])
    b_ref[...] = _dot(xn, w1b_n[...])


def _proj2_body(x_ref, w1a, w1b, a_ref, b_ref):
    x = x_ref[...]
    a_ref[...] = _dot(x, w1a[...])
    b_ref[...] = _dot(x, w1b[...])


def _style_body(d_ref, w1, b1, w2, b2, o_ref):
    h = _dot(d_ref[...], w1[...]) + b1[...]
    h = 0.5 * h * (1.0 + lax.erf(h / jnp.sqrt(2.0).astype(jnp.float32)))
    o_ref[...] = _dot(h, w2[...]) + b2[...]


def _dec_body(x_ref, w, b, o_ref):
    o_ref[...] = _dot(x_ref[...], w[...]) + b[...]


def _full_spec(arr):
    return pl.BlockSpec(arr.shape, lambda i: (0,) * arr.ndim)


def _row_spec(tile, k):
    return pl.BlockSpec((tile, k), lambda i: (i, 0))


def _mlp3(x, p, tile):
    rows, k = x.shape
    out_dim = p["W3"].shape[1]
    ws = [p["W1"], p["b1"].reshape(1, -1), p["g1"].reshape(1, -1),
          p["be1"].reshape(1, -1), p["W2"], p["b2"].reshape(1, -1),
          p["g2"].reshape(1, -1), p["be2"].reshape(1, -1), p["W3"],
          p["b3"].reshape(1, -1)]
    return pl.pallas_call(
        _mlp3_body,
        grid=(rows // tile,),
        in_specs=[_row_spec(tile, k)] + [_full_spec(w) for w in ws],
        out_specs=_row_spec(tile, out_dim),
        out_shape=jax.ShapeDtypeStruct((rows, out_dim), jnp.float32),
    )(x, *ws)


def _edge_mlp(ar, bc, e, p):
    rows = ar.shape[0]
    w1c = p["W1"][2 * D:]
    ws = [w1c, p["b1"].reshape(1, -1), p["g1"].reshape(1, -1),
          p["be1"].reshape(1, -1), p["W2"], p["b2"].reshape(1, -1),
          p["g2"].reshape(1, -1), p["be2"].reshape(1, -1), p["W3"],
          p["b3"].reshape(1, -1)]
    return pl.pallas_call(
        _edge_body,
        grid=(rows // TILE_E,),
        in_specs=[_row_spec(TILE_E, D)] * 3 + [_full_spec(w) for w in ws],
        out_specs=_row_spec(TILE_E, D),
        out_shape=jax.ShapeDtypeStruct((rows, D), jnp.float32),
    )(ar, bc, e, *ws)


def _node_mlp(x, msgp, p, w1a_n, w1b_n):
    rows = x.shape[0]
    w1x, w1m = p["W1"][:D], p["W1"][D:]
    ws = [w1x, w1m, p["b1"].reshape(1, -1), p["g1"].reshape(1, -1),
          p["be1"].reshape(1, -1), p["W2"], p["b2"].reshape(1, -1),
          p["g2"].reshape(1, -1), p["be2"].reshape(1, -1), p["W3"],
          p["b3"].reshape(1, -1), w1a_n, w1b_n]
    return pl.pallas_call(
        _node_body,
        grid=(rows // TILE_N,),
        in_specs=[_row_spec(TILE_N, D),
                  pl.BlockSpec((2, TILE_N, D), lambda i: (0, i, 0))]
                 + [_full_spec(w) for w in ws],
        out_specs=[_row_spec(TILE_N, D)] * 3,
        out_shape=[jax.ShapeDtypeStruct((rows, D), jnp.float32)] * 3,
    )(x, msgp, *ws)


def _proj2(x, w1a, w1b):
    rows = x.shape[0]
    return pl.pallas_call(
        _proj2_body,
        grid=(rows // TILE_N,),
        in_specs=[_row_spec(TILE_N, D), _full_spec(w1a), _full_spec(w1b)],
        out_specs=[_row_spec(TILE_N, D)] * 2,
        out_shape=[jax.ShapeDtypeStruct((rows, D), jnp.float32)] * 2,
    )(x, w1a, w1b)


def _style_proj(dino, w1, b1, w2, b2):
    ws = [w1, b1.reshape(1, -1), w2, b2.reshape(1, -1)]
    return pl.pallas_call(
        _style_body,
        grid=(1,),
        in_specs=[_full_spec(dino)] + [_full_spec(w) for w in ws],
        out_specs=pl.BlockSpec((dino.shape[0], D), lambda i: (0, 0)),
        out_shape=jax.ShapeDtypeStruct((dino.shape[0], D), jnp.float32),
    )(dino, *ws)


def _decode(x, w, b):
    rows = x.shape[0]
    wp = jnp.pad(w, ((0, 0), (0, D - w.shape[1])))
    bp = jnp.pad(b, (0, D - b.shape[0])).reshape(1, -1)
    out = pl.pallas_call(
        _dec_body,
        grid=(rows // TILE_N,),
        in_specs=[_row_spec(TILE_N, D), _full_spec(wp), _full_spec(bp)],
        out_specs=_row_spec(TILE_N, D),
        out_shape=jax.ShapeDtypeStruct((rows, D), jnp.float32),
    )(x, wp, bp)
    return out[:, :w.shape[1]]


def _make_gather(n, kj):
    e_pad = NW * kj * CH
    mesh = plsc.VectorSubcoreMesh(core_axis_name="c", subcore_axis_name="s")

    @functools.partial(
        pl.kernel, mesh=mesh,
        out_type=[jax.ShapeDtypeStruct((e_pad, D), jnp.float32),
                  jax.ShapeDtypeStruct((e_pad, D), jnp.float32)],
        scratch_types=[
            pltpu.VMEM((kj, CH), jnp.int32),
            pltpu.VMEM((kj, CH), jnp.int32),
            pltpu.VMEM((2 * CH, D), jnp.float32),
            pltpu.VMEM((2 * CH, D), jnp.float32),
            pltpu.SemaphoreType.DMA,
            pltpu.SemaphoreType.DMA,
        ],
    )
    def gather2(a_hbm, b_hbm, row_hbm, col_hbm, ar_hbm, bc_hbm,
                ridx, cidx, rbuf, cbuf, rsem, csem):
        c = lax.axis_index("c")
        s = lax.axis_index("s")
        wid = s * 2 + c
        pltpu.sync_copy(row_hbm.at[wid], ridx)
        pltpu.sync_copy(col_hbm.at[wid], cidx)
        base = wid * kj * CH

        def body(t, carry):
            j = t * 2
            g0 = pltpu.async_copy(a_hbm.at[ridx.at[j]],
                                  rbuf.at[pl.ds(0, CH)], rsem)
            g1 = pltpu.async_copy(a_hbm.at[ridx.at[j + 1]],
                                  rbuf.at[pl.ds(CH, CH)], rsem)
            g2 = pltpu.async_copy(b_hbm.at[cidx.at[j]],
                                  cbuf.at[pl.ds(0, CH)], csem)
            g3 = pltpu.async_copy(b_hbm.at[cidx.at[j + 1]],
                                  cbuf.at[pl.ds(CH, CH)], csem)
            g0.wait()
            g1.wait()
            pltpu.sync_copy(rbuf, ar_hbm.at[pl.ds(base + j * CH, 2 * CH)])
            g2.wait()
            g3.wait()
            pltpu.sync_copy(cbuf, bc_hbm.at[pl.ds(base + j * CH, 2 * CH)])
            return carry

        lax.fori_loop(0, kj // 2, body, 0)

    return gather2


def _make_scatter(n_acc, kj):
    mesh = plsc.VectorSubcoreMesh(core_axis_name="c", subcore_axis_name="s")
    rows_per_sub = n_acc // 16

    @functools.partial(
        pl.kernel, mesh=mesh,
        out_type=jax.ShapeDtypeStruct((2, n_acc, D), jnp.float32),
        scratch_types=[
            pltpu.VMEM((kj, CH), jnp.int32),
            pltpu.VMEM((CH, D), jnp.float32),
            pltpu.VMEM((CH, D), jnp.float32),
            pltpu.VMEM_SHARED((n_acc, D), jnp.float32),
            pltpu.SemaphoreType.DMA,
            pltpu.SemaphoreType.DMA,
            pltpu.SemaphoreType.DMA,
            pltpu.SemaphoreType.DMA,
        ],
    )
    def scatter_add(e_hbm, col_hbm, zeros_hbm, out_hbm,
                    cidx, eb0, eb1, acc, es0, es1, as0, as1):
        c = lax.axis_index("c")
        s = lax.axis_index("s")
        wid = s * 2 + c
        pltpu.sync_copy(zeros_hbm.at[pl.ds(s * rows_per_sub, rows_per_sub)],
                        acc.at[pl.ds(s * rows_per_sub, rows_per_sub)])
        plsc.subcore_barrier()
        pltpu.sync_copy(col_hbm.at[wid], cidx)
        base = wid * kj * CH

        c0 = pltpu.async_copy(e_hbm.at[pl.ds(base, CH)], eb0, es0)
        c1 = pltpu.async_copy(e_hbm.at[pl.ds(base + CH, CH)], eb1, es1)
        c0.wait()
        pltpu.async_copy(eb0, acc.at[cidx.at[0]], as0, add=True)
        c1.wait()
        pltpu.async_copy(eb1, acc.at[cidx.at[1]], as1, add=True)

        def body(t, carry):
            j = t * 2
            pltpu.make_async_copy(e_hbm.at[pl.ds(base, CH)], eb0,
                                  as0).wait()
            c0 = pltpu.async_copy(e_hbm.at[pl.ds(base + j * CH, CH)],
                                  eb0, es0)
            pltpu.make_async_copy(e_hbm.at[pl.ds(base, CH)], eb1,
                                  as1).wait()
            c1 = pltpu.async_copy(e_hbm.at[pl.ds(base + (j + 1) * CH, CH)],
                                  eb1, es1)
            c0.wait()
            pltpu.async_copy(eb0, acc.at[cidx.at[j]], as0, add=True)
            c1.wait()
            pltpu.async_copy(eb1, acc.at[cidx.at[j + 1]], as1, add=True)
            return carry

        lax.fori_loop(1, kj // 2, body, 0)
        pltpu.make_async_copy(e_hbm.at[pl.ds(base, CH)], eb0, as0).wait()
        pltpu.make_async_copy(e_hbm.at[pl.ds(base, CH)], eb1, as1).wait()
        plsc.subcore_barrier()
        pltpu.sync_copy(acc.at[pl.ds(s * rows_per_sub, rows_per_sub)],
                        out_hbm.at[c, pl.ds(s * rows_per_sub, rows_per_sub)])

    return scatter_add


def kernel(params, dino_feat, pos, smpl, mat, edge_attr, edge_index, batch):
    n = pos.shape[0]
    e_cnt = edge_attr.shape[0]
    kj = -(-e_cnt // (NW * CH))
    kj += kj % 2
    e_pad = NW * kj * CH
    n_acc = -(-(n + 1) // 128) * 128

    style = _style_proj(dino_feat, params["proj_W1"], params["proj_b1"],
                        params["proj_W2"], params["proj_b2"])

    x_in = jnp.concatenate(
        [pos, style[batch], smpl[batch], mat[batch]], axis=-1)
    k_in = x_in.shape[1]
    k_pad = -(-k_in // 8) * 8
    x_in = jnp.pad(x_in, ((0, 0), (0, k_pad - k_in)))
    ne = dict(params["node_enc"])
    ne["W1"] = jnp.pad(ne["W1"], ((0, k_pad - k_in), (0, 0)))
    x = _mlp3(x_in, ne, TILE_N)

    row = edge_index[0]
    col = edge_index[1]
    pad = e_pad - e_cnt
    row_g = jnp.pad(row, (0, pad)).reshape(NW, kj, CH)
    col_g = jnp.pad(col, (0, pad)).reshape(NW, kj, CH)
    col_s = jnp.pad(col, (0, pad), constant_values=n).reshape(NW, kj, CH)
    e = _mlp3(jnp.pad(edge_attr, ((0, pad), (0, 0))),
              params["edge_enc"], TILE_E)
    zeros_acc = jnp.zeros((n_acc, D), jnp.float32)

    gather2 = _make_gather(n, kj)
    scatter_add = _make_scatter(n_acc, kj)

    blocks = params["blocks"]
    a16, b16 = _proj2(x, blocks[0]["edge_mlp"]["W1"][:D],
                      blocks[0]["edge_mlp"]["W1"][D:2 * D])

    for i, blk in enumerate(blocks):
        ar, bc = gather2(a16, b16, row_g, col_g)
        e = _edge_mlp(ar, bc, e, blk["edge_mlp"])
        msgp = scatter_add(e, col_s, zeros_acc)
        nxt = blocks[i + 1]["edge_mlp"]["W1"] if i + 1 < len(blocks) \
            else blk["edge_mlp"]["W1"]
        x, a16, b16 = _node_mlp(x, msgp, blk["node_mlp"],
                                nxt[:D], nxt[D:2 * D])

    return _decode(x, params["dec_W"], params["dec_b"])


# speedup vs baseline: 1.4053x; 1.0257x over previous
import functools

import jax
import jax.numpy as jnp
from jax import lax
from jax.experimental import pallas as pl
from jax.experimental.pallas import tpu as pltpu
from jax.experimental.pallas import tpu_sc as plsc

D = 128
NW = 32
CH = 128
TILE_E = 2048
TILE_N = 2000


def _ln_relu(h, g, b):
    m = jnp.mean(h, axis=-1, keepdims=True)
    v = jnp.mean(jnp.square(h - m), axis=-1, keepdims=True)
    return jnp.maximum((h - m) * lax.rsqrt(v + 1e-5) * g + b, 0.0)


def _dot(a, w):
    return jnp.dot(a, w, preferred_element_type=jnp.float32)


def _mlp3_body(x_ref, w1, b1, g1, be1, w2, b2, g2, be2, w3, b3, o_ref):
    h = _dot(x_ref[...], w1[...]) + b1[...]
    h = _ln_relu(h, g1[...], be1[...])
    h = _dot(h, w2[...]) + b2[...]
    h = _ln_relu(h, g2[...], be2[...])
    o_ref[...] = _dot(h, w3[...]) + b3[...]


def _edge_body(ar_ref, bc_ref, e_ref, w1c, b1, g1, be1,
               w2, b2, g2, be2, w3, b3, o_ref):
    e = e_ref[...]
    h = ar_ref[...] + bc_ref[...] + _dot(e, w1c[...]) + b1[...]
    h = _ln_relu(h, g1[...], be1[...])
    h = _dot(h, w2[...]) + b2[...]
    h = _ln_relu(h, g2[...], be2[...])
    o_ref[...] = e + _dot(h, w3[...]) + b3[...]


def _node_body(x_ref, m_ref, w1x, w1m, b1, g1, be1,
               w2, b2, g2, be2, w3, b3, w1a_n, w1b_n,
               o_ref, a_ref, b_ref):
    x = x_ref[...]
    msg = m_ref[0] + m_ref[1]
    h = _dot(x, w1x[...]) + _dot(msg, w1m[...]) + b1[...]
    h = _ln_relu(h, g1[...], be1[...])
    h = _dot(h, w2[...]) + b2[...]
    h = _ln_relu(h, g2[...], be2[...])
    xn = x + _dot(h, w3[...]) + b3[...]
    o_ref[...] = xn
    a_ref[...] = _dot(xn, w1a_n[...])
    b_ref[...] = _dot(xn, w1b_n[...])


def _proj2_body(x_ref, w1a, w1b, a_ref, b_ref):
    x = x_ref[...]
    a_ref[...] = _dot(x, w1a[...])
    b_ref[...] = _dot(x, w1b[...])


def _style_body(d_ref, w1, b1, w2, b2, o_ref):
    h = _dot(d_ref[...], w1[...]) + b1[...]
    h = 0.5 * h * (1.0 + lax.erf(h / jnp.sqrt(2.0).astype(jnp.float32)))
    o_ref[...] = _dot(h, w2[...]) + b2[...]


def _dec_body(x_ref, w, b, o_ref):
    o_ref[...] = _dot(x_ref[...], w[...]) + b[...]


def _full_spec(arr):
    return pl.BlockSpec(arr.shape, lambda i: (0,) * arr.ndim)


def _row_spec(tile, k):
    return pl.BlockSpec((tile, k), lambda i: (i, 0))


def _mlp3(x, p, tile):
    rows, k = x.shape
    out_dim = p["W3"].shape[1]
    ws = [p["W1"], p["b1"].reshape(1, -1), p["g1"].reshape(1, -1),
          p["be1"].reshape(1, -1), p["W2"], p["b2"].reshape(1, -1),
          p["g2"].reshape(1, -1), p["be2"].reshape(1, -1), p["W3"],
          p["b3"].reshape(1, -1)]
    return pl.pallas_call(
        _mlp3_body,
        grid=(rows // tile,),
        in_specs=[_row_spec(tile, k)] + [_full_spec(w) for w in ws],
        out_specs=_row_spec(tile, out_dim),
        out_shape=jax.ShapeDtypeStruct((rows, out_dim), jnp.float32),
    )(x, *ws)


def _edge_mlp(ar, bc, e, p):
    rows = ar.shape[0]
    w1c = p["W1"][2 * D:]
    ws = [w1c, p["b1"].reshape(1, -1), p["g1"].reshape(1, -1),
          p["be1"].reshape(1, -1), p["W2"], p["b2"].reshape(1, -1),
          p["g2"].reshape(1, -1), p["be2"].reshape(1, -1), p["W3"],
          p["b3"].reshape(1, -1)]
    return pl.pallas_call(
        _edge_body,
        grid=(rows // TILE_E,),
        in_specs=[_row_spec(TILE_E, D)] * 3 + [_full_spec(w) for w in ws],
        out_specs=_row_spec(TILE_E, D),
        out_shape=jax.ShapeDtypeStruct((rows, D), jnp.float32),
    )(ar, bc, e, *ws)


def _node_mlp(x, msgp, p, w1a_n, w1b_n):
    rows = x.shape[0]
    w1x, w1m = p["W1"][:D], p["W1"][D:]
    ws = [w1x, w1m, p["b1"].reshape(1, -1), p["g1"].reshape(1, -1),
          p["be1"].reshape(1, -1), p["W2"], p["b2"].reshape(1, -1),
          p["g2"].reshape(1, -1), p["be2"].reshape(1, -1), p["W3"],
          p["b3"].reshape(1, -1), w1a_n, w1b_n]
    return pl.pallas_call(
        _node_body,
        grid=(rows // TILE_N,),
        in_specs=[_row_spec(TILE_N, D),
                  pl.BlockSpec((2, TILE_N, D), lambda i: (0, i, 0))]
                 + [_full_spec(w) for w in ws],
        out_specs=[_row_spec(TILE_N, D)] * 3,
        out_shape=[jax.ShapeDtypeStruct((rows, D), jnp.float32)] * 3,
    )(x, msgp, *ws)


def _proj2(x, w1a, w1b):
    rows = x.shape[0]
    return pl.pallas_call(
        _proj2_body,
        grid=(rows // TILE_N,),
        in_specs=[_row_spec(TILE_N, D), _full_spec(w1a), _full_spec(w1b)],
        out_specs=[_row_spec(TILE_N, D)] * 2,
        out_shape=[jax.ShapeDtypeStruct((rows, D), jnp.float32)] * 2,
    )(x, w1a, w1b)


def _style_proj(dino, w1, b1, w2, b2):
    ws = [w1, b1.reshape(1, -1), w2, b2.reshape(1, -1)]
    return pl.pallas_call(
        _style_body,
        grid=(1,),
        in_specs=[_full_spec(dino)] + [_full_spec(w) for w in ws],
        out_specs=pl.BlockSpec((dino.shape[0], D), lambda i: (0, 0)),
        out_shape=jax.ShapeDtypeStruct((dino.shape[0], D), jnp.float32),
    )(dino, *ws)


def _decode(x, w, b):
    rows = x.shape[0]
    wp = jnp.pad(w, ((0, 0), (0, D - w.shape[1])))
    bp = jnp.pad(b, (0, D - b.shape[0])).reshape(1, -1)
    out = pl.pallas_call(
        _dec_body,
        grid=(rows // TILE_N,),
        in_specs=[_row_spec(TILE_N, D), _full_spec(wp), _full_spec(bp)],
        out_specs=_row_spec(TILE_N, D),
        out_shape=jax.ShapeDtypeStruct((rows, D), jnp.float32),
    )(x, wp, bp)
    return out[:, :w.shape[1]]


def _make_gather(n, kj):
    e_pad = NW * kj * CH
    mesh = plsc.VectorSubcoreMesh(core_axis_name="c", subcore_axis_name="s")

    @functools.partial(
        pl.kernel, mesh=mesh,
        out_type=[jax.ShapeDtypeStruct((e_pad, D), jnp.float32),
                  jax.ShapeDtypeStruct((e_pad, D), jnp.float32)],
        scratch_types=[
            pltpu.VMEM((kj, CH), jnp.int32),
            pltpu.VMEM((kj, CH), jnp.int32),
            pltpu.VMEM((CH, D), jnp.float32),
            pltpu.VMEM((CH, D), jnp.float32),
            pltpu.VMEM((CH, D), jnp.float32),
            pltpu.VMEM((CH, D), jnp.float32),
            pltpu.SemaphoreType.DMA,
            pltpu.SemaphoreType.DMA,
            pltpu.SemaphoreType.DMA,
            pltpu.SemaphoreType.DMA,
            pltpu.SemaphoreType.DMA,
            pltpu.SemaphoreType.DMA,
            pltpu.SemaphoreType.DMA,
            pltpu.SemaphoreType.DMA,
        ],
    )
    def gather2(a_hbm, b_hbm, row_hbm, col_hbm, ar_hbm, bc_hbm,
                ridx, cidx, ra, rb, ca, cb,
                gra, grb, gca, gcb, wra, wrb, wca, wcb):
        c = lax.axis_index("c")
        s = lax.axis_index("s")
        wid = s * 2 + c
        pltpu.sync_copy(row_hbm.at[wid], ridx)
        pltpu.sync_copy(col_hbm.at[wid], cidx)
        base = wid * kj * CH

        pltpu.async_copy(a_hbm.at[ridx.at[0]], ra, gra)
        pltpu.async_copy(a_hbm.at[ridx.at[1]], rb, grb)
        pltpu.async_copy(b_hbm.at[cidx.at[0]], ca, gca)
        pltpu.async_copy(b_hbm.at[cidx.at[1]], cb, gcb)

        def body(t, carry):
            j = t * 2
            pltpu.make_async_copy(a_hbm.at[ridx.at[j]], ra, gra).wait()
            pltpu.async_copy(ra, ar_hbm.at[pl.ds(base + (j - 2) * CH, CH)],
                             wra)
            pltpu.make_async_copy(a_hbm.at[ridx.at[j]], rb, grb).wait()
            pltpu.async_copy(rb, ar_hbm.at[pl.ds(base + (j - 1) * CH, CH)],
                             wrb)
            pltpu.make_async_copy(a_hbm.at[ridx.at[j]], ca, gca).wait()
            pltpu.async_copy(ca, bc_hbm.at[pl.ds(base + (j - 2) * CH, CH)],
                             wca)
            pltpu.make_async_copy(a_hbm.at[ridx.at[j]], cb, gcb).wait()
            pltpu.async_copy(cb, bc_hbm.at[pl.ds(base + (j - 1) * CH, CH)],
                             wcb)
            pltpu.make_async_copy(a_hbm.at[ridx.at[j]], ra, wra).wait()
            pltpu.async_copy(a_hbm.at[ridx.at[j]], ra, gra)
            pltpu.make_async_copy(a_hbm.at[ridx.at[j]], rb, wrb).wait()
            pltpu.async_copy(a_hbm.at[ridx.at[j + 1]], rb, grb)
            pltpu.make_async_copy(a_hbm.at[ridx.at[j]], ca, wca).wait()
            pltpu.async_copy(b_hbm.at[cidx.at[j]], ca, gca)
            pltpu.make_async_copy(a_hbm.at[ridx.at[j]], cb, wcb).wait()
            pltpu.async_copy(b_hbm.at[cidx.at[j + 1]], cb, gcb)
            return carry

        lax.fori_loop(1, kj // 2, body, 0)

        pltpu.make_async_copy(a_hbm.at[ridx.at[0]], ra, gra).wait()
        pltpu.sync_copy(ra, ar_hbm.at[pl.ds(base + (kj - 2) * CH, CH)])
        pltpu.make_async_copy(a_hbm.at[ridx.at[0]], rb, grb).wait()
        pltpu.sync_copy(rb, ar_hbm.at[pl.ds(base + (kj - 1) * CH, CH)])
        pltpu.make_async_copy(a_hbm.at[ridx.at[0]], ca, gca).wait()
        pltpu.sync_copy(ca, bc_hbm.at[pl.ds(base + (kj - 2) * CH, CH)])
        pltpu.make_async_copy(a_hbm.at[ridx.at[0]], cb, gcb).wait()
        pltpu.sync_copy(cb, bc_hbm.at[pl.ds(base + (kj - 1) * CH, CH)])

    return gather2


def _make_scatter(n_acc, kj):
    mesh = plsc.VectorSubcoreMesh(core_axis_name="c", subcore_axis_name="s")
    rows_per_sub = n_acc // 16

    @functools.partial(
        pl.kernel, mesh=mesh,
        out_type=jax.ShapeDtypeStruct((2, n_acc, D), jnp.float32),
        scratch_types=[
            pltpu.VMEM((kj, CH), jnp.int32),
            pltpu.VMEM((CH, D), jnp.float32),
            pltpu.VMEM((CH, D), jnp.float32),
            pltpu.VMEM_SHARED((n_acc, D), jnp.float32),
            pltpu.SemaphoreType.DMA,
            pltpu.SemaphoreType.DMA,
            pltpu.SemaphoreType.DMA,
            pltpu.SemaphoreType.DMA,
        ],
    )
    def scatter_add(e_hbm, col_hbm, zeros_hbm, out_hbm,
                    cidx, eb0, eb1, acc, es0, es1, as0, as1):
        c = lax.axis_index("c")
        s = lax.axis_index("s")
        wid = s * 2 + c
        pltpu.sync_copy(zeros_hbm.at[pl.ds(s * rows_per_sub, rows_per_sub)],
                        acc.at[pl.ds(s * rows_per_sub, rows_per_sub)])
        plsc.subcore_barrier()
        pltpu.sync_copy(col_hbm.at[wid], cidx)
        base = wid * kj * CH

        c0 = pltpu.async_copy(e_hbm.at[pl.ds(base, CH)], eb0, es0)
        c1 = pltpu.async_copy(e_hbm.at[pl.ds(base + CH, CH)], eb1, es1)
        c0.wait()
        pltpu.async_copy(eb0, acc.at[cidx.at[0]], as0, add=True)
        c1.wait()
        pltpu.async_copy(eb1, acc.at[cidx.at[1]], as1, add=True)

        def body(t, carry):
            j = t * 2
            pltpu.make_async_copy(e_hbm.at[pl.ds(base, CH)], eb0,
                                  as0).wait()
            c0 = pltpu.async_copy(e_hbm.at[pl.ds(base + j * CH, CH)],
                                  eb0, es0)
            pltpu.make_async_copy(e_hbm.at[pl.ds(base, CH)], eb1,
                                  as1).wait()
            c1 = pltpu.async_copy(e_hbm.at[pl.ds(base + (j + 1) * CH, CH)],
                                  eb1, es1)
            c0.wait()
            pltpu.async_copy(eb0, acc.at[cidx.at[j]], as0, add=True)
            c1.wait()
            pltpu.async_copy(eb1, acc.at[cidx.at[j + 1]], as1, add=True)
            return carry

        lax.fori_loop(1, kj // 2, body, 0)
        pltpu.make_async_copy(e_hbm.at[pl.ds(base, CH)], eb0, as0).wait()
        pltpu.make_async_copy(e_hbm.at[pl.ds(base, CH)], eb1, as1).wait()
        plsc.subcore_barrier()
        pltpu.sync_copy(acc.at[pl.ds(s * rows_per_sub, rows_per_sub)],
                        out_hbm.at[c, pl.ds(s * rows_per_sub, rows_per_sub)])

    return scatter_add


def kernel(params, dino_feat, pos, smpl, mat, edge_attr, edge_index, batch):
    n = pos.shape[0]
    e_cnt = edge_attr.shape[0]
    kj = -(-e_cnt // (NW * CH))
    kj += kj % 2
    e_pad = NW * kj * CH
    n_acc = -(-(n + 1) // 128) * 128

    style = _style_proj(dino_feat, params["proj_W1"], params["proj_b1"],
                        params["proj_W2"], params["proj_b2"])

    x_in = jnp.concatenate(
        [pos, style[batch], smpl[batch], mat[batch]], axis=-1)
    k_in = x_in.shape[1]
    k_pad = -(-k_in // 8) * 8
    x_in = jnp.pad(x_in, ((0, 0), (0, k_pad - k_in)))
    ne = dict(params["node_enc"])
    ne["W1"] = jnp.pad(ne["W1"], ((0, k_pad - k_in), (0, 0)))
    x = _mlp3(x_in, ne, TILE_N)

    row = edge_index[0]
    col = edge_index[1]
    pad = e_pad - e_cnt
    row_g = jnp.pad(row, (0, pad)).reshape(NW, kj, CH)
    col_g = jnp.pad(col, (0, pad)).reshape(NW, kj, CH)
    col_s = jnp.pad(col, (0, pad), constant_values=n).reshape(NW, kj, CH)
    e = _mlp3(jnp.pad(edge_attr, ((0, pad), (0, 0))),
              params["edge_enc"], TILE_E)
    zeros_acc = jnp.zeros((n_acc, D), jnp.float32)

    gather2 = _make_gather(n, kj)
    scatter_add = _make_scatter(n_acc, kj)

    blocks = params["blocks"]
    a16, b16 = _proj2(x, blocks[0]["edge_mlp"]["W1"][:D],
                      blocks[0]["edge_mlp"]["W1"][D:2 * D])

    for i, blk in enumerate(blocks):
        ar, bc = gather2(a16, b16, row_g, col_g)
        e = _edge_mlp(ar, bc, e, blk["edge_mlp"])
        msgp = scatter_add(e, col_s, zeros_acc)
        nxt = blocks[i + 1]["edge_mlp"]["W1"] if i + 1 < len(blocks) \
            else blk["edge_mlp"]["W1"]
        x, a16, b16 = _node_mlp(x, msgp, blk["node_mlp"],
                                nxt[:D], nxt[D:2 * D])

    return _decode(x, params["dec_W"], params["dec_b"])
